# baseline (speedup 1.0000x reference)
import functools

import jax
import jax.numpy as jnp
from jax import lax
from jax.experimental import pallas as pl
from jax.experimental.pallas import tpu as pltpu
from jax.experimental.pallas import tpu_sc as plsc

B = 16384
D = 64
NC = 2
NS = 16
NW = NC * NS
BPW = B // NW


def _sc_gather_body(uid_hbm, eid_hbm, uemb_hbm, eemb_hbm, ub_hbm, eb_hbm,
                    ue_out, ee_out, ub_out, eb_out,
                    uidx_v, eidx_v, urows_v, erows_v, ubv, ebv, sem):
    wid = lax.axis_index("s") * NC + lax.axis_index("c")
    base = wid * BPW
    pltpu.sync_copy(uid_hbm.at[pl.ds(base, BPW)], uidx_v)
    pltpu.sync_copy(eid_hbm.at[pl.ds(base, BPW)], eidx_v)
    cu = pltpu.async_copy(uemb_hbm.at[uidx_v], urows_v, sem)
    ce = pltpu.async_copy(eemb_hbm.at[eidx_v], erows_v, sem)
    cub = pltpu.async_copy(ub_hbm.at[uidx_v], ubv, sem)
    ceb = pltpu.async_copy(eb_hbm.at[eidx_v], ebv, sem)
    cu.wait()
    ce.wait()
    cub.wait()
    ceb.wait()
    pltpu.sync_copy(urows_v, ue_out.at[pl.ds(base, BPW)])
    pltpu.sync_copy(erows_v, ee_out.at[pl.ds(base, BPW)])
    pltpu.sync_copy(ubv, ub_out.at[pl.ds(base, BPW)])
    pltpu.sync_copy(ebv, eb_out.at[pl.ds(base, BPW)])


@functools.cache
def _sc_gather():
    return pl.kernel(
        _sc_gather_body,
        out_type=[
            jax.ShapeDtypeStruct((B, D), jnp.float32),
            jax.ShapeDtypeStruct((B, D), jnp.float32),
            jax.ShapeDtypeStruct((B, 1), jnp.float32),
            jax.ShapeDtypeStruct((B, 1), jnp.float32),
        ],
        mesh=plsc.VectorSubcoreMesh(core_axis_name="c", subcore_axis_name="s"),
        compiler_params=pltpu.CompilerParams(use_tc_tiling_on_sc=False),
        scratch_types=[
            pltpu.VMEM((BPW,), jnp.int32),
            pltpu.VMEM((BPW,), jnp.int32),
            pltpu.VMEM((BPW, D), jnp.float32),
            pltpu.VMEM((BPW, D), jnp.float32),
            pltpu.VMEM((BPW, 1), jnp.float32),
            pltpu.VMEM((BPW, 1), jnp.float32),
            pltpu.SemaphoreType.DMA,
        ],
    )


def _mlp_body(ue_ref, ee_ref, ub_ref, eb_ref, w1a_ref, w1b_ref, b1_ref,
              g1_ref, be1_ref, w2_ref, b2_ref, g2_ref, be2_ref, w3_ref,
              b3_ref, w4_ref, b4gb_ref, out_ref):
    ue = ue_ref[...]
    ee = ee_ref[...]
    h = (jnp.dot(ue, w1a_ref[...], preferred_element_type=jnp.float32)
         + jnp.dot(ee, w1b_ref[...], preferred_element_type=jnp.float32)
         + b1_ref[...])
    h = jnp.maximum(h, 0.0)
    m = jnp.mean(h, axis=0, keepdims=True)
    v = jnp.mean(jnp.square(h - m), axis=0, keepdims=True)
    h = (h - m) * lax.rsqrt(v + 1e-5) * g1_ref[...] + be1_ref[...]
    h = jnp.maximum(jnp.dot(h, w2_ref[...], preferred_element_type=jnp.float32)
                    + b2_ref[...], 0.0)
    m = jnp.mean(h, axis=0, keepdims=True)
    v = jnp.mean(jnp.square(h - m), axis=0, keepdims=True)
    h = (h - m) * lax.rsqrt(v + 1e-5) * g2_ref[...] + be2_ref[...]
    h = jnp.maximum(jnp.dot(h, w3_ref[...], preferred_element_type=jnp.float32)
                    + b3_ref[...], 0.0)
    mlp_out = jnp.sum(h * w4_ref[...], axis=1, keepdims=True)
    mf = jnp.sum(ue * ee, axis=1, keepdims=True) + ub_ref[...] + eb_ref[...]
    out = mlp_out + 0.3 * mf + b4gb_ref[0, 0]
    out_ref[...] = jax.nn.sigmoid(out)


_mlp = pl.pallas_call(
    _mlp_body,
    out_shape=jax.ShapeDtypeStruct((B, 1), jnp.float32),
)


def kernel(user_ids, exercise_ids, user_emb, ex_emb, user_b, ex_b, global_b,
           W1, b1, g1, be1, W2, b2, g2, be2, W3, b3, W4, b4):
    uid = user_ids.astype(jnp.int32)
    eid = exercise_ids.astype(jnp.int32)
    ue, ee, ub, eb = _sc_gather()(uid, eid, user_emb, ex_emb, user_b, ex_b)
    w1a = W1[:, :D].T
    w1b = W1[:, D:].T
    b4gb = (0.7 * b4 + 0.3 * global_b).reshape(1, 1)
    return _mlp(ue, ee, ub, eb, w1a, w1b, b1.reshape(1, -1),
                g1.reshape(1, -1), be1.reshape(1, -1), W2.T,
                b2.reshape(1, -1), g2.reshape(1, -1), be2.reshape(1, -1),
                W3.T, b3.reshape(1, -1), W4.reshape(1, D) * 0.7, b4gb)
